# baseline (speedup 1.0000x reference)
import functools

import jax
import jax.numpy as jnp
from jax import lax
from jax.experimental import pallas as pl
from jax.experimental.pallas import tpu as pltpu
from jax.experimental.pallas import tpu_sc as plsc

_VOCAB = 1000
_D = 1000
_BATCH = 1024
_SEQ = 50
_NTV = 125
_NC, _NS = 2, 16
_NW = _NC * _NS
_TPW = 4


@functools.partial(
    pl.kernel,
    mesh=plsc.VectorSubcoreMesh(core_axis_name="c", subcore_axis_name="s"),
    out_type=jax.ShapeDtypeStruct((_SEQ, _NTV, 8, 8, 128), jnp.float32),
    scratch_types=[
        pltpu.VMEM((_TPW, 8 * _VOCAB), jnp.float32),
        pltpu.VMEM((2, _TPW, 8, 8, 128), jnp.float32),
        pltpu.VMEM((2, _BATCH), jnp.int32),
        pltpu.SemaphoreType.DMA((2, _TPW)),
        pltpu.SemaphoreType.DMA((2,)),
    ],
    compiler_params=pltpu.CompilerParams(
        use_tc_tiling_on_sc=False, needs_layout_passes=False
    ),
)
def _gather(table_hbm, idx_hbm, out_hbm, tcols, obuf, ibuf, osem, isem):
    wid = lax.axis_index("s") * _NC + lax.axis_index("c")
    tv0 = wid * _TPW
    pltpu.sync_copy(table_hbm.at[pl.ds(tv0, _TPW)], tcols)

    def idx_dma(s, db):
        return pltpu.make_async_copy(
            idx_hbm.at[s], ibuf.at[db], isem.at[db]
        )

    def out_dma(s, db, t):
        return pltpu.make_async_copy(
            obuf.at[db, t],
            out_hbm.at[s, tv0 + t],
            osem.at[db, t],
        )

    idx_dma(0, 0).start()

    def do_s(s, db):
        idx_dma(s, db).wait()

        @pl.when(s + 1 < _SEQ)
        def _():
            idx_dma(s + 1, 1 - db).start()

        @pl.when(s >= 2)
        def _():
            for t in range(_TPW):
                @pl.when(tv0 + t < _NTV)
                def _():
                    out_dma(s - 2, db, t).wait()

        @plsc.parallel_loop(0, 64, unroll=4)
        def _g_body(g):
            tb = g // 8
            gg = g % 8
            iv = ibuf[db, pl.ds(16 * g, 16)]
            vals = []
            for sub in range(8):
                fidx = iv + jnp.int32(_VOCAB * sub)
                for t in range(_TPW):
                    vals.append(
                        (sub, t, plsc.load_gather(tcols.at[t], [fidx]))
                    )
            for sub, t, v in vals:
                obuf[db, t, tb, sub, pl.ds(16 * gg, 16)] = v

        for t in range(_TPW):
            @pl.when(tv0 + t < _NTV)
            def _():
                out_dma(s, db, t).start()

    def outer(s2, carry):
        for db in range(2):
            do_s(s2 * 2 + db, db)
        return carry

    lax.fori_loop(0, _SEQ // 2, outer, 0)

    for db in range(2):
        for t in range(_TPW):
            @pl.when(tv0 + t < _NTV)
            def _():
                out_dma(_SEQ - 2 + db, db, t).wait()


def kernel(idx, table):
    idx_t = jnp.transpose(idx).astype(jnp.int32)
    table_t = jnp.transpose(table).reshape(_NTV, 8 * _VOCAB)
    table_p = jnp.pad(table_t, ((0, _NW * _TPW - _NTV), (0, 0)))
    out5d = _gather(table_p, idx_t)
    t = jnp.transpose(out5d, (2, 4, 0, 1, 3))
    return t.reshape(_BATCH, _SEQ, _D)


# speedup vs baseline: 1.0425x; 1.0425x over previous
import functools

import jax
import jax.numpy as jnp
from jax import lax
from jax.experimental import pallas as pl
from jax.experimental.pallas import tpu as pltpu
from jax.experimental.pallas import tpu_sc as plsc

_VOCAB = 1000
_D = 1000
_BATCH = 1024
_SEQ = 50
_NTV = 125
_NC, _NS = 2, 16
_NW = _NC * _NS
_TPW = 4


@functools.partial(
    pl.kernel,
    mesh=plsc.VectorSubcoreMesh(core_axis_name="c", subcore_axis_name="s"),
    out_type=jax.ShapeDtypeStruct((_SEQ, _NTV, 8, 8, 128), jnp.float32),
    scratch_types=[
        pltpu.VMEM((_TPW, 8 * _VOCAB), jnp.float32),
        pltpu.VMEM((2, _TPW, 8, 8, 128), jnp.float32),
        pltpu.VMEM((2, _BATCH), jnp.int32),
        pltpu.SemaphoreType.DMA((2, _TPW)),
        pltpu.SemaphoreType.DMA((2,)),
    ],
    compiler_params=pltpu.CompilerParams(
        use_tc_tiling_on_sc=False, needs_layout_passes=False
    ),
)
def _gather(table_hbm, idx_hbm, out_hbm, tcols, obuf, ibuf, osem, isem):
    wid = lax.axis_index("s") * _NC + lax.axis_index("c")
    tv0 = wid * _TPW
    pltpu.sync_copy(table_hbm.at[pl.ds(tv0, _TPW)], tcols)

    def idx_dma(s, db):
        return pltpu.make_async_copy(
            idx_hbm.at[s], ibuf.at[db], isem.at[db]
        )

    def out_dma(s, db, t):
        return pltpu.make_async_copy(
            obuf.at[db, t],
            out_hbm.at[s, tv0 + t],
            osem.at[db, t],
        )

    idx_dma(0, 0).start()

    def do_s(s, db):
        idx_dma(s, db).wait()

        @pl.when(s + 1 < _SEQ)
        def _():
            idx_dma(s + 1, 1 - db).start()

        @pl.when(s >= 2)
        def _():
            for t in range(_TPW):
                @pl.when(tv0 + t < _NTV)
                def _():
                    out_dma(s - 2, db, t).wait()

        @plsc.parallel_loop(0, 64, unroll=2)
        def _g_body(g):
            tb = g // 8
            gg = g % 8
            iv = ibuf[db, pl.ds(16 * g, 16)]
            vals = []
            for sub in range(8):
                fidx = iv + jnp.int32(_VOCAB * sub)
                for t in range(_TPW):
                    vals.append(
                        (sub, t, plsc.load_gather(tcols.at[t], [fidx]))
                    )
            for sub, t, v in vals:
                obuf[db, t, tb, sub, pl.ds(16 * gg, 16)] = v

        for t in range(_TPW):
            @pl.when(tv0 + t < _NTV)
            def _():
                out_dma(s, db, t).start()

    def outer(s2, carry):
        for db in range(2):
            do_s(s2 * 2 + db, db)
        return carry

    lax.fori_loop(0, _SEQ // 2, outer, 0)

    for db in range(2):
        for t in range(_TPW):
            @pl.when(tv0 + t < _NTV)
            def _():
                out_dma(_SEQ - 2 + db, db, t).wait()


def kernel(idx, table):
    idx_t = jnp.transpose(idx).astype(jnp.int32)
    table_t = jnp.transpose(table).reshape(_NTV, 8 * _VOCAB)
    table_p = jnp.pad(table_t, ((0, _NW * _TPW - _NTV), (0, 0)))
    out5d = _gather(table_p, idx_t)
    t = jnp.transpose(out5d, (2, 4, 0, 1, 3))
    return t.reshape(_BATCH, _SEQ, _D)
